# baseline (speedup 1.0000x reference)
import functools

import jax
import jax.numpy as jnp
from jax import lax
from jax.experimental import pallas as pl
from jax.experimental.pallas import tpu as pltpu
from jax.experimental.pallas import tpu_sc as plsc

OUT_H = 7
OUT_W = 7
SPATIAL_SCALE = 0.25
SAMPLING_RATIO = 2
LANES = 16

G_BINS = 1
NBUF = 2

_GDN = lax.GatherDimensionNumbers(
    offset_dims=(), collapsed_slice_dims=(0,), start_index_map=(0,))
_PIB = lax.GatherScatterMode.PROMISE_IN_BOUNDS


def _lane_broadcast(vec, i):
    return lax.gather(vec, jnp.full((LANES, 1), i, jnp.int32), _GDN, (1,),
                      mode=_PIB)


def _axis_samples(start, bin_size, pooled, grid, size):
    p = jnp.arange(pooled, dtype=jnp.float32)
    g = jnp.arange(grid, dtype=jnp.float32)
    coord = (start[:, None, None]
             + p[None, :, None] * bin_size[:, None, None]
             + (g[None, None, :] + 0.5) * bin_size[:, None, None] / grid)
    valid = (coord >= -1.0) & (coord <= float(size))
    c = jnp.maximum(coord, 0.0)
    low = jnp.floor(c).astype(jnp.int32)
    cond = low >= size - 1
    low = jnp.where(cond, size - 1, low)
    high = jnp.where(cond, size - 1, low + 1)
    cc = jnp.where(cond, low.astype(jnp.float32), c)
    l = cc - low.astype(jnp.float32)
    h = 1.0 - l
    m = valid.astype(jnp.float32)
    idx = jnp.concatenate([low, high], axis=-1)
    w = jnp.concatenate([h * m, l * m], axis=-1)
    return idx, w


def _bin_indices_weights(rois, B, H, W):
    offset = 0.5
    bidx = rois[:, 0].astype(jnp.int32)
    sw = rois[:, 1] * SPATIAL_SCALE - offset
    sh = rois[:, 2] * SPATIAL_SCALE - offset
    ew = rois[:, 3] * SPATIAL_SCALE - offset
    eh = rois[:, 4] * SPATIAL_SCALE - offset
    bh = (eh - sh) / OUT_H
    bw = (ew - sw) / OUT_W
    yi, wy = _axis_samples(sh, bh, OUT_H, SAMPLING_RATIO, H)
    xi, wx = _axis_samples(sw, bw, OUT_W, SAMPLING_RATIO, W)
    yb = bidx[:, None, None] * H + yi
    idx = yb[:, :, None, :, None] * W + xi[:, None, :, None, :]
    w = wy[:, :, None, :, None] * wx[:, None, :, None, :] * 0.25
    R = rois.shape[0]
    return (idx.reshape(R * OUT_H * OUT_W * 16).astype(jnp.int32),
            w.reshape(R * OUT_H * OUT_W * 16))


def _sc_pool(table, idx, wts, n_bins):
    C = table.shape[1]
    info = plsc.get_sparse_core_info()
    nw = info.num_cores * info.num_subcores
    bins_per_w = n_bins // nw
    steps = bins_per_w // G_BINS
    mesh = plsc.VectorSubcoreMesh(core_axis_name="c", subcore_axis_name="s")

    @functools.partial(
        pl.kernel,
        mesh=mesh,
        out_type=jax.ShapeDtypeStruct((n_bins, C), jnp.float32),
        scratch_types=[
            pltpu.VMEM((bins_per_w * 16,), jnp.int32),
            pltpu.VMEM((bins_per_w * 16,), jnp.float32),
            pltpu.VMEM((NBUF, G_BINS * 16, C), jnp.float32),
            pltpu.VMEM((NBUF, G_BINS, C), jnp.float32),
            pltpu.SemaphoreType.DMA,
            pltpu.SemaphoreType.DMA,
        ],
    )
    def k(table_hbm, idx_hbm, wts_hbm, out_hbm, idx_v, wts_v, rows_v, out_v,
          gsem, osem):
        wid = lax.axis_index("s") * info.num_cores + lax.axis_index("c")
        ibase = wid * bins_per_w * 16
        obase = wid * bins_per_w

        pltpu.sync_copy(idx_hbm.at[pl.ds(ibase, bins_per_w * 16)], idx_v)
        pltpu.sync_copy(wts_hbm.at[pl.ds(ibase, bins_per_w * 16)], wts_v)

        def gather(ch, buf):
            return pltpu.make_async_copy(
                table_hbm.at[idx_v.at[pl.ds(ch * (G_BINS * 16), G_BINS * 16)]],
                rows_v.at[buf], gsem)

        def out_desc(buf):
            return pltpu.make_async_copy(
                out_v.at[buf], out_hbm.at[pl.ds(obase, G_BINS)], osem)

        for b in range(NBUF - 1):
            gather(b, b).start()

        def body(g, carry):
            for p in range(NBUF):
                ch = NBUF * g + p

                @pl.when(g >= 1)
                def _():
                    out_desc(p).wait()

                @pl.when(ch + NBUF - 1 < steps)
                def _():
                    gather(ch + NBUF - 1, (p + NBUF - 1) % NBUF).start()

                gather(ch, p).wait()
                for b in range(G_BINS):
                    wv = wts_v[pl.ds((ch * G_BINS + b) * 16, 16)]
                    ws = [_lane_broadcast(wv, i) for i in range(16)]
                    for c in range(C // LANES):
                        acc = ws[0] * rows_v[p, b * 16, pl.ds(c * LANES, LANES)]
                        for i in range(1, 16):
                            acc = acc + ws[i] * rows_v[p, b * 16 + i,
                                                       pl.ds(c * LANES, LANES)]
                        out_v[p, b, pl.ds(c * LANES, LANES)] = acc
                pltpu.async_copy(
                    out_v.at[p],
                    out_hbm.at[pl.ds(obase + ch * G_BINS, G_BINS)], osem)
            return carry

        lax.fori_loop(0, steps // NBUF, body, 0)
        for b in range(NBUF):
            out_desc(b).wait()

    return k(table, idx, wts)


def kernel(input, rois):
    B, C, H, W = input.shape
    R = rois.shape[0]
    table = jnp.transpose(input, (0, 2, 3, 1)).reshape(B * H * W, C)
    idx, wts = _bin_indices_weights(rois, B, H, W)
    out = _sc_pool(table, idx, wts, R * OUT_H * OUT_W)
    return jnp.transpose(out.reshape(R, OUT_H, OUT_W, C), (0, 3, 1, 2))


# speedup vs baseline: 1.0535x; 1.0535x over previous
import functools

import jax
import jax.numpy as jnp
from jax import lax
from jax.experimental import pallas as pl
from jax.experimental.pallas import tpu as pltpu
from jax.experimental.pallas import tpu_sc as plsc

OUT_H = 7
OUT_W = 7
SPATIAL_SCALE = 0.25
SAMPLING_RATIO = 2
LANES = 16

G_BINS = 2
NBUF = 4

_GDN = lax.GatherDimensionNumbers(
    offset_dims=(), collapsed_slice_dims=(0,), start_index_map=(0,))
_PIB = lax.GatherScatterMode.PROMISE_IN_BOUNDS


def _lane_broadcast(vec, i):
    return lax.gather(vec, jnp.full((LANES, 1), i, jnp.int32), _GDN, (1,),
                      mode=_PIB)


def _axis_samples(start, bin_size, pooled, grid, size):
    p = jnp.arange(pooled, dtype=jnp.float32)
    g = jnp.arange(grid, dtype=jnp.float32)
    coord = (start[:, None, None]
             + p[None, :, None] * bin_size[:, None, None]
             + (g[None, None, :] + 0.5) * bin_size[:, None, None] / grid)
    valid = (coord >= -1.0) & (coord <= float(size))
    c = jnp.maximum(coord, 0.0)
    low = jnp.floor(c).astype(jnp.int32)
    cond = low >= size - 1
    low = jnp.where(cond, size - 1, low)
    high = jnp.where(cond, size - 1, low + 1)
    cc = jnp.where(cond, low.astype(jnp.float32), c)
    l = cc - low.astype(jnp.float32)
    h = 1.0 - l
    m = valid.astype(jnp.float32)
    idx = jnp.concatenate([low, high], axis=-1)
    w = jnp.concatenate([h * m, l * m], axis=-1)
    return idx, w


def _bin_indices_weights(rois, B, H, W):
    offset = 0.5
    bidx = rois[:, 0].astype(jnp.int32)
    sw = rois[:, 1] * SPATIAL_SCALE - offset
    sh = rois[:, 2] * SPATIAL_SCALE - offset
    ew = rois[:, 3] * SPATIAL_SCALE - offset
    eh = rois[:, 4] * SPATIAL_SCALE - offset
    bh = (eh - sh) / OUT_H
    bw = (ew - sw) / OUT_W
    yi, wy = _axis_samples(sh, bh, OUT_H, SAMPLING_RATIO, H)
    xi, wx = _axis_samples(sw, bw, OUT_W, SAMPLING_RATIO, W)
    yb = bidx[:, None, None] * H + yi
    idx = yb[:, :, None, :, None] * W + xi[:, None, :, None, :]
    w = wy[:, :, None, :, None] * wx[:, None, :, None, :] * 0.25
    R = rois.shape[0]
    return (idx.reshape(R * OUT_H * OUT_W * 16).astype(jnp.int32),
            w.reshape(R * OUT_H * OUT_W * 16))


def _sc_pool(table, idx, wts, n_bins):
    C = table.shape[1]
    info = plsc.get_sparse_core_info()
    nw = info.num_cores * info.num_subcores
    bins_per_w = n_bins // nw
    steps = bins_per_w // G_BINS
    mesh = plsc.VectorSubcoreMesh(core_axis_name="c", subcore_axis_name="s")

    @functools.partial(
        pl.kernel,
        mesh=mesh,
        out_type=jax.ShapeDtypeStruct((n_bins, C), jnp.float32),
        scratch_types=[
            pltpu.VMEM((bins_per_w * 16,), jnp.int32),
            pltpu.VMEM((bins_per_w * 16,), jnp.float32),
            pltpu.VMEM((NBUF, G_BINS * 16, C), jnp.float32),
            pltpu.VMEM((NBUF, G_BINS, C), jnp.float32),
            pltpu.SemaphoreType.DMA,
            pltpu.SemaphoreType.DMA,
        ],
    )
    def k(table_hbm, idx_hbm, wts_hbm, out_hbm, idx_v, wts_v, rows_v, out_v,
          gsem, osem):
        wid = lax.axis_index("s") * info.num_cores + lax.axis_index("c")
        ibase = wid * bins_per_w * 16
        obase = wid * bins_per_w

        pltpu.sync_copy(idx_hbm.at[pl.ds(ibase, bins_per_w * 16)], idx_v)
        pltpu.sync_copy(wts_hbm.at[pl.ds(ibase, bins_per_w * 16)], wts_v)

        def gather(ch, buf):
            return pltpu.make_async_copy(
                table_hbm.at[idx_v.at[pl.ds(ch * (G_BINS * 16), G_BINS * 16)]],
                rows_v.at[buf], gsem)

        def out_desc(buf):
            return pltpu.make_async_copy(
                out_v.at[buf], out_hbm.at[pl.ds(obase, G_BINS)], osem)

        for b in range(NBUF - 1):
            gather(b, b).start()

        def body(g, carry):
            for p in range(NBUF):
                ch = NBUF * g + p

                @pl.when(g >= 1)
                def _():
                    out_desc(p).wait()

                @pl.when(ch + NBUF - 1 < steps)
                def _():
                    gather(ch + NBUF - 1, (p + NBUF - 1) % NBUF).start()

                gather(ch, p).wait()
                for b in range(G_BINS):
                    wv = wts_v[pl.ds((ch * G_BINS + b) * 16, 16)]
                    ws = [_lane_broadcast(wv, i) for i in range(16)]
                    for c in range(C // LANES):
                        acc = ws[0] * rows_v[p, b * 16, pl.ds(c * LANES, LANES)]
                        for i in range(1, 16):
                            acc = acc + ws[i] * rows_v[p, b * 16 + i,
                                                       pl.ds(c * LANES, LANES)]
                        out_v[p, b, pl.ds(c * LANES, LANES)] = acc
                pltpu.async_copy(
                    out_v.at[p],
                    out_hbm.at[pl.ds(obase + ch * G_BINS, G_BINS)], osem)
            return carry

        lax.fori_loop(0, steps // NBUF, body, 0)
        for b in range(NBUF):
            out_desc(b).wait()

    return k(table, idx, wts)


def kernel(input, rois):
    B, C, H, W = input.shape
    R = rois.shape[0]
    table = jnp.transpose(input, (0, 2, 3, 1)).reshape(B * H * W, C)
    idx, wts = _bin_indices_weights(rois, B, H, W)
    out = _sc_pool(table, idx, wts, R * OUT_H * OUT_W)
    return jnp.transpose(out.reshape(R, OUT_H, OUT_W, C), (0, 3, 1, 2))


# speedup vs baseline: 1.2499x; 1.1864x over previous
import functools

import jax
import jax.numpy as jnp
from jax import lax
from jax.experimental import pallas as pl
from jax.experimental.pallas import tpu as pltpu
from jax.experimental.pallas import tpu_sc as plsc

OUT_H = 7
OUT_W = 7
SPATIAL_SCALE = 0.25
SAMPLING_RATIO = 2
LANES = 16

G_BINS = 2
NBUF = 2

_GDN = lax.GatherDimensionNumbers(
    offset_dims=(), collapsed_slice_dims=(0,), start_index_map=(0,))
_PIB = lax.GatherScatterMode.PROMISE_IN_BOUNDS


def _lane_broadcast(vec, i):
    return lax.gather(vec, jnp.full((LANES, 1), i, jnp.int32), _GDN, (1,),
                      mode=_PIB)


def _axis_samples(start, bin_size, pooled, grid, size):
    p = jnp.arange(pooled, dtype=jnp.float32)
    g = jnp.arange(grid, dtype=jnp.float32)
    coord = (start[:, None, None]
             + p[None, :, None] * bin_size[:, None, None]
             + (g[None, None, :] + 0.5) * bin_size[:, None, None] / grid)
    valid = (coord >= -1.0) & (coord <= float(size))
    c = jnp.maximum(coord, 0.0)
    low = jnp.floor(c).astype(jnp.int32)
    cond = low >= size - 1
    low = jnp.where(cond, size - 1, low)
    high = jnp.where(cond, size - 1, low + 1)
    cc = jnp.where(cond, low.astype(jnp.float32), c)
    l = cc - low.astype(jnp.float32)
    h = 1.0 - l
    m = valid.astype(jnp.float32)
    idx = jnp.concatenate([low, high], axis=-1)
    w = jnp.concatenate([h * m, l * m], axis=-1)
    return idx, w


def _bin_indices_weights(rois, B, H, W):
    offset = 0.5
    bidx = rois[:, 0].astype(jnp.int32)
    sw = rois[:, 1] * SPATIAL_SCALE - offset
    sh = rois[:, 2] * SPATIAL_SCALE - offset
    ew = rois[:, 3] * SPATIAL_SCALE - offset
    eh = rois[:, 4] * SPATIAL_SCALE - offset
    bh = (eh - sh) / OUT_H
    bw = (ew - sw) / OUT_W
    yi, wy = _axis_samples(sh, bh, OUT_H, SAMPLING_RATIO, H)
    xi, wx = _axis_samples(sw, bw, OUT_W, SAMPLING_RATIO, W)
    yb = bidx[:, None, None] * H + yi
    idx = yb[:, :, None, :, None] * W + xi[:, None, :, None, :]
    w = wy[:, :, None, :, None] * wx[:, None, :, None, :] * 0.25
    R = rois.shape[0]
    return (idx.reshape(R * OUT_H * OUT_W * 16).astype(jnp.int32),
            w.reshape(R * OUT_H * OUT_W * 16))


def _sc_pool(table, idx, wts, n_bins):
    C = table.shape[1]
    info = plsc.get_sparse_core_info()
    nw = info.num_cores * info.num_subcores
    bins_per_w = n_bins // nw
    steps = bins_per_w // G_BINS
    mesh = plsc.VectorSubcoreMesh(core_axis_name="c", subcore_axis_name="s")

    @functools.partial(
        pl.kernel,
        mesh=mesh,
        out_type=jax.ShapeDtypeStruct((n_bins, C), jnp.float32),
        scratch_types=[
            pltpu.VMEM((bins_per_w * 16,), jnp.int32),
            pltpu.VMEM((bins_per_w * 16,), jnp.float32),
            pltpu.VMEM((NBUF, G_BINS * 16, C), jnp.float32),
            pltpu.VMEM((NBUF, G_BINS, C), jnp.float32),
            pltpu.SemaphoreType.DMA,
            pltpu.SemaphoreType.DMA,
        ],
    )
    def k(table_hbm, idx_hbm, wts_hbm, out_hbm, idx_v, wts_v, rows_v, out_v,
          gsem, osem):
        wid = lax.axis_index("s") * info.num_cores + lax.axis_index("c")
        ibase = wid * bins_per_w * 16
        obase = wid * bins_per_w

        pltpu.sync_copy(idx_hbm.at[pl.ds(ibase, bins_per_w * 16)], idx_v)
        pltpu.sync_copy(wts_hbm.at[pl.ds(ibase, bins_per_w * 16)], wts_v)

        def gather(ch, buf):
            return pltpu.make_async_copy(
                table_hbm.at[idx_v.at[pl.ds(ch * (G_BINS * 16), G_BINS * 16)]],
                rows_v.at[buf], gsem)

        def out_desc(buf):
            return pltpu.make_async_copy(
                out_v.at[buf], out_hbm.at[pl.ds(obase, G_BINS)], osem)

        for b in range(NBUF - 1):
            gather(b, b).start()

        def body(g, carry):
            for p in range(NBUF):
                ch = NBUF * g + p

                @pl.when(g >= 1)
                def _():
                    out_desc(p).wait()

                @pl.when(ch + NBUF - 1 < steps)
                def _():
                    gather(ch + NBUF - 1, (p + NBUF - 1) % NBUF).start()

                gather(ch, p).wait()
                for b in range(G_BINS):
                    wv = wts_v[pl.ds((ch * G_BINS + b) * 16, 16)]
                    ws = [_lane_broadcast(wv, i) for i in range(16)]
                    for c in range(C // LANES):
                        acc = ws[0] * rows_v[p, b * 16, pl.ds(c * LANES, LANES)]
                        for i in range(1, 16):
                            acc = acc + ws[i] * rows_v[p, b * 16 + i,
                                                       pl.ds(c * LANES, LANES)]
                        out_v[p, b, pl.ds(c * LANES, LANES)] = acc
                pltpu.async_copy(
                    out_v.at[p],
                    out_hbm.at[pl.ds(obase + ch * G_BINS, G_BINS)], osem)
            return carry

        lax.fori_loop(0, steps // NBUF, body, 0)
        for b in range(NBUF):
            out_desc(b).wait()

    return k(table, idx, wts)


def kernel(input, rois):
    B, C, H, W = input.shape
    R = rois.shape[0]
    table = jnp.transpose(input, (0, 2, 3, 1)).reshape(B * H * W, C)
    idx, wts = _bin_indices_weights(rois, B, H, W)
    out = _sc_pool(table, idx, wts, R * OUT_H * OUT_W)
    return jnp.transpose(out.reshape(R, OUT_H, OUT_W, C), (0, 3, 1, 2))
